# baseline (speedup 1.0000x reference)
import functools

import jax
import jax.numpy as jnp
from jax import lax
from jax.experimental import pallas as pl
from jax.experimental.pallas import tpu as pltpu
from jax.experimental.pallas import tpu_sc as plsc

_LANES = 16
_CHUNK = 128


def _features_linear_sc(B, F, NW, NC):
    bpw = B // NW
    n_idx = F * bpw
    n_gather = n_idx // _CHUNK

    mesh = plsc.VectorSubcoreMesh(core_axis_name="c", subcore_axis_name="s")

    @functools.partial(
        pl.kernel,
        mesh=mesh,
        compiler_params=pltpu.CompilerParams(needs_layout_passes=False),
        out_type=jax.ShapeDtypeStruct((B,), jnp.float32),
        scratch_types=[
            pltpu.VMEM((n_idx,), jnp.int32),
            pltpu.VMEM((n_idx,), jnp.float32),
            pltpu.VMEM((bpw,), jnp.float32),
            pltpu.VMEM((_LANES,), jnp.float32),
            pltpu.SemaphoreType.DMA,
        ],
    )
    def run(xr_hbm, t_hbm, b_hbm, out_hbm, idx_v, rows_v, out_v, bias_v, sem):
        wid = lax.axis_index("s") * NC + lax.axis_index("c")

        pltpu.sync_copy(xr_hbm.at[wid], idx_v)
        pltpu.sync_copy(b_hbm, bias_v.at[pl.ds(0, 1)])

        def gather_body(g, carry):
            off = g * _CHUNK
            pltpu.async_copy(
                t_hbm.at[idx_v.at[pl.ds(off, _CHUNK)]],
                rows_v.at[pl.ds(off, _CHUNK)],
                sem,
            )
            return carry

        lax.fori_loop(0, n_gather, gather_body, 0)
        pltpu.make_async_copy(t_hbm.at[pl.ds(0, n_idx)], rows_v, sem).wait()

        b0 = bias_v[pl.ds(0, _LANES)][0]
        lane_f = lax.iota(jnp.int32, _LANES) * F

        def acc_body(t, carry):
            base = t * _LANES * F
            acc = jnp.zeros((_LANES,), jnp.float32) + b0
            for f in range(F):
                acc = acc + plsc.load_gather(rows_v, [lane_f + (base + f)])
            out_v[pl.ds(t * _LANES, _LANES)] = acc
            return carry

        lax.fori_loop(0, bpw // _LANES, acc_body, 0)

        pltpu.sync_copy(out_v, out_hbm.at[pl.ds(wid * bpw, bpw)])

    return run


def kernel(x, table, bias):
    B, F = x.shape
    V, D = table.shape
    assert D == 1

    info = plsc.get_sparse_core_info()
    NC, NS = info.num_cores, info.num_subcores
    NW = NC * NS

    bpw = B // NW
    assert B % (NW * _LANES) == 0 and (F * bpw) % _CHUNK == 0

    xr = x.astype(jnp.int32).reshape(NW, F * bpw)
    t_flat = table.reshape(-1)

    out = _features_linear_sc(B, F, NW, NC)(xr, t_flat, bias)
    return out.reshape(B, 1)


# speedup vs baseline: 1.1063x; 1.1063x over previous
import functools

import jax
import jax.numpy as jnp
from jax import lax
from jax.experimental import pallas as pl
from jax.experimental.pallas import tpu as pltpu
from jax.experimental.pallas import tpu_sc as plsc

_LANES = 16
_CHUNK = 128


def _features_linear_sc(B, F, NW, NC):
    bpw = B // NW
    n_idx = F * bpw
    n_gather = n_idx // _CHUNK

    mesh = plsc.VectorSubcoreMesh(core_axis_name="c", subcore_axis_name="s")

    @functools.partial(
        pl.kernel,
        mesh=mesh,
        out_type=jax.ShapeDtypeStruct((B,), jnp.float32),
        scratch_types=[
            pltpu.VMEM((n_idx,), jnp.int32),
            pltpu.VMEM((n_idx,), jnp.float32),
            pltpu.VMEM((bpw,), jnp.float32),
            pltpu.VMEM((_LANES,), jnp.float32),
            pltpu.SemaphoreType.DMA,
        ],
    )
    def run(xr_hbm, t_hbm, b_hbm, out_hbm, idx_v, rows_v, out_v, bias_v, sem):
        wid = lax.axis_index("s") * NC + lax.axis_index("c")

        pltpu.sync_copy(xr_hbm.at[wid], idx_v)
        pltpu.sync_copy(b_hbm, bias_v.at[pl.ds(0, 1)])

        def gather_body(g, carry):
            off = g * _CHUNK
            pltpu.async_copy(
                t_hbm.at[idx_v.at[pl.ds(off, _CHUNK)]],
                rows_v.at[pl.ds(off, _CHUNK)],
                sem,
            )
            return carry

        lax.fori_loop(0, n_gather, gather_body, 0)
        pltpu.make_async_copy(t_hbm.at[pl.ds(0, n_idx)], rows_v, sem).wait()

        b0 = bias_v[pl.ds(0, _LANES)][0]

        def acc_body(t, carry):
            off = t * _LANES
            acc = jnp.zeros((_LANES,), jnp.float32) + b0
            for f in range(F):
                acc = acc + rows_v[pl.ds(f * bpw + off, _LANES)]
            out_v[pl.ds(off, _LANES)] = acc
            return carry

        lax.fori_loop(0, bpw // _LANES, acc_body, 0)

        pltpu.sync_copy(out_v, out_hbm.at[pl.ds(wid * bpw, bpw)])

    return run


def kernel(x, table, bias):
    B, F = x.shape
    V, D = table.shape
    assert D == 1

    info = plsc.get_sparse_core_info()
    NC, NS = info.num_cores, info.num_subcores
    NW = NC * NS

    bpw = B // NW
    assert B % (NW * _LANES) == 0 and (F * bpw) % _CHUNK == 0

    xr = (
        x.astype(jnp.int32)
        .T.reshape(F, NW, bpw)
        .transpose(1, 0, 2)
        .reshape(NW, F * bpw)
    )
    t_flat = table.T.reshape(-1)

    out = _features_linear_sc(B, F, NW, NC)(xr, t_flat, bias)
    return out.reshape(B, 1)


# speedup vs baseline: 2.0915x; 1.8905x over previous
import functools

import jax
import jax.numpy as jnp
from jax import lax
from jax.experimental import pallas as pl
from jax.experimental.pallas import tpu as pltpu
from jax.experimental.pallas import tpu_sc as plsc

_LANES = 16
_CHUNK = 128


def _features_linear_sc(B, F, NW, NC):
    bpw = B // NW
    n_idx = F * bpw
    n_gather = n_idx // _CHUNK

    mesh = plsc.VectorSubcoreMesh(core_axis_name="c", subcore_axis_name="s")

    @functools.partial(
        pl.kernel,
        mesh=mesh,
        out_type=jax.ShapeDtypeStruct((B,), jnp.float32),
        scratch_types=[
            pltpu.VMEM((n_idx,), jnp.int32),
            pltpu.VMEM((n_idx,), jnp.float32),
            pltpu.VMEM((bpw,), jnp.float32),
            pltpu.VMEM((_LANES,), jnp.float32),
            pltpu.SemaphoreType.DMA,
        ],
    )
    def run(xr_hbm, t_hbm, b_hbm, out_hbm, idx_v, rows_v, out_v, bias_v, sem):
        wid = lax.axis_index("s") * NC + lax.axis_index("c")

        pltpu.sync_copy(xr_hbm.at[wid], idx_v)
        pltpu.sync_copy(b_hbm, bias_v.at[pl.ds(0, 1)])

        def gather_body(g, carry):
            off = g * _CHUNK
            pltpu.async_copy(
                t_hbm.at[idx_v.at[pl.ds(off, _CHUNK)]],
                rows_v.at[pl.ds(off, _CHUNK)],
                sem,
            )
            return carry

        lax.fori_loop(0, n_gather, gather_body, 0)
        pltpu.make_async_copy(t_hbm.at[pl.ds(0, n_idx)], rows_v, sem).wait()

        b0 = bias_v[pl.ds(0, _LANES)][0]

        def acc_body(t, carry):
            off = t * _LANES
            acc = jnp.zeros((_LANES,), jnp.float32) + b0
            for f in range(F):
                acc = acc + rows_v[pl.ds(f * bpw + off, _LANES)]
            out_v[pl.ds(off, _LANES)] = acc
            return carry

        lax.fori_loop(0, bpw // _LANES, acc_body, 0)

        pltpu.sync_copy(out_v, out_hbm.at[pl.ds(wid * bpw, bpw)])

    return run


def kernel(x, table, bias):
    B, F = x.shape
    V, D = table.shape
    assert D == 1

    info = plsc.get_sparse_core_info()
    NC, NS = info.num_cores, info.num_subcores
    NW = NC * NS

    bpw = B // NW
    assert B % (NW * _LANES) == 0 and (F * bpw) % _CHUNK == 0

    xr = (
        x.astype(jnp.int32)
        .T.reshape(F, NW, bpw)
        .transpose(1, 0, 2)
        .reshape(NW, F * bpw)
    )
    vh = V // 1024 * 1024
    t_flat = jnp.concatenate(
        [table[:vh].reshape(-1), table[vh:].reshape(-1)]
    )

    out = _features_linear_sc(B, F, NW, NC)(xr, t_flat, bias)
    return out.reshape(B, 1)


# speedup vs baseline: 2.0953x; 1.0018x over previous
import functools

import jax
import jax.numpy as jnp
from jax import lax
from jax.experimental import pallas as pl
from jax.experimental.pallas import tpu as pltpu
from jax.experimental.pallas import tpu_sc as plsc

_LANES = 16
_CHUNK = 1024


def _features_linear_sc(B, F, NW, NC):
    bpw = B // NW
    n_idx = F * bpw
    n_gather = n_idx // _CHUNK

    mesh = plsc.VectorSubcoreMesh(core_axis_name="c", subcore_axis_name="s")

    @functools.partial(
        pl.kernel,
        mesh=mesh,
        out_type=jax.ShapeDtypeStruct((B,), jnp.float32),
        scratch_types=[
            pltpu.VMEM((n_idx,), jnp.int32),
            pltpu.VMEM((n_idx,), jnp.float32),
            pltpu.VMEM((bpw,), jnp.float32),
            pltpu.VMEM((_LANES,), jnp.float32),
            pltpu.SemaphoreType.DMA,
        ],
    )
    def run(xr_hbm, t_hbm, b_hbm, out_hbm, idx_v, rows_v, out_v, bias_v, sem):
        wid = lax.axis_index("s") * NC + lax.axis_index("c")

        pltpu.sync_copy(xr_hbm.at[wid], idx_v)
        pltpu.sync_copy(b_hbm, bias_v.at[pl.ds(0, 1)])

        def gather_body(g, carry):
            off = g * _CHUNK
            pltpu.async_copy(
                t_hbm.at[idx_v.at[pl.ds(off, _CHUNK)]],
                rows_v.at[pl.ds(off, _CHUNK)],
                sem,
            )
            return carry

        lax.fori_loop(0, n_gather, gather_body, 0)
        pltpu.make_async_copy(t_hbm.at[pl.ds(0, n_idx)], rows_v, sem).wait()

        b0 = bias_v[pl.ds(0, _LANES)][0]

        def acc_body(t, carry):
            off = t * _LANES
            acc = jnp.zeros((_LANES,), jnp.float32) + b0
            for f in range(F):
                acc = acc + rows_v[pl.ds(f * bpw + off, _LANES)]
            out_v[pl.ds(off, _LANES)] = acc
            return carry

        lax.fori_loop(0, bpw // _LANES, acc_body, 0)

        pltpu.sync_copy(out_v, out_hbm.at[pl.ds(wid * bpw, bpw)])

    return run


def kernel(x, table, bias):
    B, F = x.shape
    V, D = table.shape
    assert D == 1

    info = plsc.get_sparse_core_info()
    NC, NS = info.num_cores, info.num_subcores
    NW = NC * NS

    bpw = B // NW
    assert B % (NW * _LANES) == 0 and (F * bpw) % _CHUNK == 0

    xr = (
        x.astype(jnp.int32)
        .T.reshape(F, NW, bpw)
        .transpose(1, 0, 2)
        .reshape(NW, F * bpw)
    )
    vh = V // 1024 * 1024
    t_flat = jnp.concatenate(
        [table[:vh].reshape(-1), table[vh:].reshape(-1)]
    )

    out = _features_linear_sc(B, F, NW, NC)(xr, t_flat, bias)
    return out.reshape(B, 1)


# speedup vs baseline: 2.3159x; 1.1053x over previous
import functools

import jax
import jax.numpy as jnp
from jax import lax
from jax.experimental import pallas as pl
from jax.experimental.pallas import tpu as pltpu
from jax.experimental.pallas import tpu_sc as plsc

_LANES = 16
_CHUNK = 1024


def _features_linear_sc(B, F, NW, NC):
    bpw = B // NW
    n_idx = F * bpw
    n_gather = n_idx // _CHUNK

    mesh = plsc.VectorSubcoreMesh(core_axis_name="c", subcore_axis_name="s")

    @functools.partial(
        pl.kernel,
        mesh=mesh,
        out_type=jax.ShapeDtypeStruct((B,), jnp.float32),
        scratch_types=[
            pltpu.VMEM((n_idx,), jnp.int32),
            pltpu.VMEM((n_idx,), jnp.float32),
            pltpu.VMEM((bpw,), jnp.float32),
            pltpu.VMEM((_LANES,), jnp.float32),
            pltpu.SemaphoreType.DMA,
        ],
    )
    def run(xt_hbm, t_hbm, b_hbm, out_hbm, idx_v, rows_v, out_v, bias_v, sem):
        wid = lax.axis_index("s") * NC + lax.axis_index("c")
        base = wid * bpw

        for f in range(F):
            pltpu.async_copy(
                xt_hbm.at[f, pl.ds(base, bpw)],
                idx_v.at[pl.ds(f * bpw, bpw)],
                sem,
            )
        pltpu.sync_copy(b_hbm, bias_v.at[pl.ds(0, 1)])
        pltpu.make_async_copy(
            xt_hbm.at[0, pl.ds(0, n_idx)], idx_v, sem
        ).wait()

        def gather_body(g, carry):
            off = g * _CHUNK
            pltpu.async_copy(
                t_hbm.at[idx_v.at[pl.ds(off, _CHUNK)]],
                rows_v.at[pl.ds(off, _CHUNK)],
                sem,
            )
            return carry

        lax.fori_loop(0, n_gather, gather_body, 0)
        pltpu.make_async_copy(t_hbm.at[pl.ds(0, n_idx)], rows_v, sem).wait()

        b0 = bias_v[pl.ds(0, _LANES)][0]

        def acc_body(t, carry):
            off = t * _LANES
            acc = jnp.zeros((_LANES,), jnp.float32) + b0
            for f in range(F):
                acc = acc + rows_v[pl.ds(f * bpw + off, _LANES)]
            out_v[pl.ds(off, _LANES)] = acc
            return carry

        lax.fori_loop(0, bpw // _LANES, acc_body, 0)

        pltpu.sync_copy(out_v, out_hbm.at[pl.ds(base, bpw)])

    return run


def kernel(x, table, bias):
    B, F = x.shape
    V, D = table.shape
    assert D == 1

    info = plsc.get_sparse_core_info()
    NC, NS = info.num_cores, info.num_subcores
    NW = NC * NS

    bpw = B // NW
    assert B % (NW * _LANES) == 0 and (F * bpw) % _CHUNK == 0

    xt = x.astype(jnp.int32).T

    vh = V // 1024 * 1024
    t_flat = jnp.concatenate(
        [table[:vh].reshape(-1), table[vh:].reshape(-1)]
    )

    out = _features_linear_sc(B, F, NW, NC)(xt, t_flat, bias)
    return out.reshape(B, 1)
